# baseline (speedup 1.0000x reference)
import functools

import jax
import jax.numpy as jnp
from jax import lax
from jax.experimental import pallas as pl
from jax.experimental.pallas import tpu as pltpu

H, A = 100, 32
HA = H * A
POP = 2048
NUM_TOPK = 205
NUM_ITERS = 5
MOMENTUM = 0.1

C = 256
NB = POP // C
LG = HA // 128


def _cumsum_lanes(x):
    n = x.shape[1]
    k = 1
    while k < n:
        shifted = jnp.pad(x, ((0, 0), (k, 0)))[:, :n]
        x = x + shifted
        k *= 2
    return x


def _values_kernel(eps_ref, loc_ref, scale_ref, tgt_ref, q4_ref, vout_ref):
    d = (loc_ref[...] + scale_ref[...] * eps_ref[...]) - tgt_ref[...]
    q4 = q4_ref[...]
    acc = jnp.zeros((C, 128), jnp.float32)
    for k in range(LG):
        dk = d[:, 128 * k:128 * (k + 1)]
        ek = lax.dot_general(dk, q4, (((1,), (0,)), ((), ())),
                             preferred_element_type=jnp.float32)
        acc = acc + ek * dk
    vout_ref[...] = -jnp.sum(acc, axis=1, keepdims=True).reshape(1, 1, C)


def _select_kernel(vals_ref, eps_ref, loc_ref, scale_ref, means0_ref,
                   nloc_ref, nscale_ref, bs_ref, bv_ref,
                   w_scr, bh_scr, acc_e, acc_e2, acc_b):
    j = pl.program_id(0)

    @pl.when(j == 0)
    def _():
        v = vals_ref[...]
        b = v.view(jnp.int32)
        key = jnp.where(b < 0, b ^ jnp.int32(0x7FFFFFFF), b)
        ukey = key.view(jnp.uint32) ^ jnp.uint32(0x80000000)

        def body(k, t):
            t_try = t | (jnp.uint32(1) << jnp.uint32(31 - k))
            cnt = jnp.sum((ukey >= t_try).astype(jnp.int32))
            return jnp.where(cnt >= NUM_TOPK, t_try, t)

        t = lax.fori_loop(0, 32, body, jnp.uint32(0))
        gt = ukey > t
        eq = ukey == t
        need = NUM_TOPK - jnp.sum(gt.astype(jnp.int32))
        c1 = _cumsum_lanes(eq.astype(jnp.float32))
        rowtot = jnp.sum(eq.astype(jnp.float32), axis=1, keepdims=True)
        tril = (lax.broadcasted_iota(jnp.int32, (NB, NB), 0)
                > lax.broadcasted_iota(jnp.int32, (NB, NB), 1))
        roff = lax.dot_general(tril.astype(jnp.float32), rowtot,
                               (((1,), (0,)), ((), ())),
                               precision=lax.Precision.HIGHEST,
                               preferred_element_type=jnp.float32)
        rank = c1 + roff
        wsel = gt | (eq & (rank <= need.astype(jnp.float32)))
        w_scr[...] = wsel.astype(jnp.float32)
        kmax = jnp.max(key)
        eqb = key == kmax
        cb = _cumsum_lanes(eqb.astype(jnp.float32))
        rowtb = jnp.sum(eqb.astype(jnp.float32), axis=1, keepdims=True)
        roffb = lax.dot_general(tril.astype(jnp.float32), rowtb,
                                (((1,), (0,)), ((), ())),
                                precision=lax.Precision.HIGHEST,
                                preferred_element_type=jnp.float32)
        bh_scr[...] = (eqb & ((cb + roffb) == 1.0)).astype(jnp.float32)
        bv_ref[...] = jnp.max(v).reshape(1, 1)
        acc_e[...] = jnp.zeros_like(acc_e)
        acc_e2[...] = jnp.zeros_like(acc_e2)
        acc_b[...] = jnp.zeros_like(acc_b)

    eps = eps_ref[...]
    wj = w_scr[pl.ds(j, 1), :]
    bhj = bh_scr[pl.ds(j, 1), :]
    dot = functools.partial(lax.dot_general,
                            dimension_numbers=(((1,), (0,)), ((), ())),
                            precision=lax.Precision.HIGHEST,
                            preferred_element_type=jnp.float32)
    acc_e[...] += dot(wj, eps)
    acc_e2[...] += dot(wj, eps * eps)
    acc_b[...] += dot(bhj, eps)

    @pl.when(j == NB - 1)
    def _():
        loc = loc_ref[...]
        scale = scale_ref[...]
        inv = jnp.float32(1.0 / NUM_TOPK)
        m_e = acc_e[...] * inv
        m_e2 = acc_e2[...] * inv
        new_means = loc + scale * m_e
        var_eps = (m_e2 - m_e * m_e) * jnp.float32(NUM_TOPK / (NUM_TOPK - 1))
        new_stds = scale * jnp.sqrt(jnp.maximum(var_eps, 0.0))
        nloc_ref[...] = (jnp.float32(MOMENTUM) * means0_ref[...]
                         + jnp.float32(1.0 - MOMENTUM) * new_means)
        nscale_ref[...] = (jnp.float32(MOMENTUM)
                           + jnp.float32(1.0 - MOMENTUM) * new_stds)
        bs_ref[...] = loc + scale * acc_b[...]


def _values_call(eps, loc, scale, tgt, q4):
    return pl.pallas_call(
        _values_kernel,
        grid=(NB,),
        in_specs=[
            pl.BlockSpec((C, HA), lambda j: (j, 0)),
            pl.BlockSpec((1, HA), lambda j: (0, 0)),
            pl.BlockSpec((1, HA), lambda j: (0, 0)),
            pl.BlockSpec((1, HA), lambda j: (0, 0)),
            pl.BlockSpec((128, 128), lambda j: (0, 0)),
        ],
        out_specs=pl.BlockSpec((1, 1, C), lambda j: (j, 0, 0)),
        out_shape=jax.ShapeDtypeStruct((NB, 1, C), jnp.float32),
    )(eps, loc, scale, tgt, q4).reshape(NB, C)


def _select_call(vals, eps, loc, scale, means0):
    return pl.pallas_call(
        _select_kernel,
        grid=(NB,),
        in_specs=[
            pl.BlockSpec((NB, C), lambda j: (0, 0)),
            pl.BlockSpec((C, HA), lambda j: (j, 0)),
            pl.BlockSpec((1, HA), lambda j: (0, 0)),
            pl.BlockSpec((1, HA), lambda j: (0, 0)),
            pl.BlockSpec((1, HA), lambda j: (0, 0)),
        ],
        out_specs=[
            pl.BlockSpec((1, HA), lambda j: (0, 0)),
            pl.BlockSpec((1, HA), lambda j: (0, 0)),
            pl.BlockSpec((1, HA), lambda j: (0, 0)),
            pl.BlockSpec((1, 1), lambda j: (0, 0)),
        ],
        out_shape=[
            jax.ShapeDtypeStruct((1, HA), jnp.float32),
            jax.ShapeDtypeStruct((1, HA), jnp.float32),
            jax.ShapeDtypeStruct((1, HA), jnp.float32),
            jax.ShapeDtypeStruct((1, 1), jnp.float32),
        ],
        scratch_shapes=[
            pltpu.VMEM((NB, C), jnp.float32),
            pltpu.VMEM((NB, C), jnp.float32),
            pltpu.VMEM((1, HA), jnp.float32),
            pltpu.VMEM((1, HA), jnp.float32),
            pltpu.VMEM((1, HA), jnp.float32),
        ],
    )(vals, eps, loc, scale, means0)


def kernel(initial_solution, target, Q):
    means0 = initial_solution.reshape(1, HA)
    tgt = target.reshape(1, HA)
    q4 = jnp.kron(jnp.eye(4, dtype=jnp.float32), Q)

    base = jax.random.key(42)
    loc = means0
    scale = jnp.ones((1, HA), jnp.float32)
    best_actions = jnp.zeros((1, HA), jnp.float32)
    best_value = jnp.float32(-jnp.inf)
    for i in range(NUM_ITERS):
        eps = jax.random.normal(jax.random.fold_in(base, i),
                                (POP, H, A), jnp.float32).reshape(POP, HA)
        vals = _values_call(eps, loc, scale, tgt, q4)
        nloc, nscale, bs, bv = _select_call(vals, eps, loc, scale, means0)
        better = bv[0, 0] > best_value
        best_value = jnp.where(better, bv[0, 0], best_value)
        best_actions = jnp.where(better, bs, best_actions)
        loc, scale = nloc, nscale
    return best_actions.reshape(H, A)
